# baseline (speedup 1.0000x reference)
import functools
import jax
import jax.numpy as jnp
from jax import lax
from jax.experimental import pallas as pl
from jax.experimental.pallas import tpu as pltpu, tpu_sc as plsc

N = 1048576
OUT_F = 6
PED_F = 4
SPA_F = 2
G = 128
NG = N // G

_info = plsc.get_sparse_core_info()
NC = _info.num_cores
NS = _info.num_subcores
L = _info.num_lanes
NW = NC * NS

GPW = NG // NW
SG = 16
STEPS = GPW // SG
S = SG * G
BLOCKS = S // L
KPG = G // L
NBUF = 3


def _body(table_hbm, sp3_hbm, ped_hbm, out_hbm, tbl_v, *scratch):
    pv = scratch[0:NBUF]
    ov = scratch[NBUF:2 * NBUF]
    psem = scratch[2 * NBUF:3 * NBUF]
    ssem = scratch[3 * NBUF:4 * NBUF]
    osem = scratch[4 * NBUF:5 * NBUF]

    wid = lax.axis_index("s") * NC + lax.axis_index("c")
    base_g = wid * GPW

    def ped_cp(s):
        b = s % NBUF
        g0 = base_g + s * SG
        return pltpu.make_async_copy(ped_hbm.at[pl.ds(g0 * G, S)], pv[b], psem[b])

    def sp_cp(s):
        b = s % NBUF
        g0 = base_g + s * SG
        return pltpu.make_async_copy(
            sp3_hbm.at[pl.ds(g0, SG), :, :],
            ov[b].at[:, pl.ds(PED_F, SPA_F), :],
            ssem[b],
        )

    def out_cp(s):
        b = s % NBUF
        g0 = base_g + s * SG
        return pltpu.make_async_copy(ov[b], out_hbm.at[pl.ds(g0, SG), :, :], osem[b])

    ped_cp(0).start()
    sp_cp(0).start()
    pltpu.sync_copy(table_hbm, tbl_v)

    for s in range(STEPS):
        b = s % NBUF
        ped_cp(s).wait()
        if s + 1 < STEPS:
            if s - 2 >= 0:
                out_cp(s - 2).wait()
            ped_cp(s + 1).start()
            sp_cp(s + 1).start()

        ped_b = pv[b]
        out_b = ov[b]

        @plsc.parallel_loop(0, BLOCKS, unroll=4)
        def blk_fn(blk):
            sg = lax.shift_right_logical(blk, 3)
            k = jnp.bitwise_and(blk, KPG - 1)
            ped_vec = ped_b[pl.ds(blk * L, L)]
            for j in range(PED_F):
                cj = jnp.full((L,), j, jnp.int32)
                col = plsc.load_gather(tbl_v, [cj, ped_vec])
                out_b[sg, j, pl.ds(k * L, L)] = col

        sp_cp(s).wait()
        out_cp(s).start()

    for s in range(max(0, STEPS - NBUF), STEPS):
        out_cp(s).wait()


def kernel(pedestal_table, spatial_embeddings, pedestals):
    mesh = plsc.VectorSubcoreMesh(core_axis_name="c", subcore_axis_name="s")
    tblT = jnp.pad(pedestal_table.T, ((0, 0), (0, G - 16)))
    sp3 = spatial_embeddings.reshape(NG, G, SPA_F).transpose(0, 2, 1)
    scratch = (
        [pltpu.VMEM((PED_F, G), jnp.float32)]
        + [pltpu.VMEM((S,), jnp.int32) for _ in range(NBUF)]
        + [pltpu.VMEM((SG, 8, G), jnp.float32) for _ in range(NBUF)]
        + [pltpu.SemaphoreType.DMA for _ in range(3 * NBUF)]
    )
    k = functools.partial(
        pl.kernel,
        mesh=mesh,
        out_type=jax.ShapeDtypeStruct((NG, 8, G), jnp.float32),
        scratch_types=scratch,
        compiler_params=pltpu.CompilerParams(
            needs_layout_passes=False,
            use_tc_tiling_on_sc=False,
            skip_device_barrier=True,
            disable_bounds_checks=True,
            disable_semaphore_checks=True,
        ),
    )(_body)
    out3 = k(tblT, sp3, pedestals)
    return out3.transpose(0, 2, 1).reshape(N, 8)[:, :OUT_F]


# speedup vs baseline: 1.1280x; 1.1280x over previous
import functools
import jax
import jax.numpy as jnp
from jax import lax
from jax.experimental import pallas as pl
from jax.experimental.pallas import tpu as pltpu, tpu_sc as plsc

N = 1048576
OUT_F = 6
PED_F = 4
SPA_F = 2
G = 128
NG = N // G

_info = plsc.get_sparse_core_info()
NC = _info.num_cores
NS = _info.num_subcores
L = _info.num_lanes
NW = NC * NS

GPW = NG // NW
SG = 32
STEPS = GPW // SG
S = SG * G
BLOCKS = S // L
KPG = G // L
NBUF = 3


def _body(table_hbm, sp3_hbm, ped_hbm, out_hbm, tbl_v, *scratch):
    pv = scratch[0:NBUF]
    ov = scratch[NBUF:2 * NBUF]
    psem = scratch[2 * NBUF:3 * NBUF]
    ssem = scratch[3 * NBUF:4 * NBUF]
    osem = scratch[4 * NBUF:5 * NBUF]

    wid = lax.axis_index("s") * NC + lax.axis_index("c")
    base_g = wid * GPW

    def ped_cp(s):
        b = s % NBUF
        g0 = base_g + s * SG
        return pltpu.make_async_copy(ped_hbm.at[pl.ds(g0 * G, S)], pv[b], psem[b])

    def sp_cp(s):
        b = s % NBUF
        g0 = base_g + s * SG
        return pltpu.make_async_copy(
            sp3_hbm.at[pl.ds(g0, SG), :, :],
            ov[b].at[:, pl.ds(PED_F, SPA_F), :],
            ssem[b],
        )

    def out_cp(s):
        b = s % NBUF
        g0 = base_g + s * SG
        return pltpu.make_async_copy(ov[b], out_hbm.at[pl.ds(g0, SG), :, :], osem[b])

    ped_cp(0).start()
    ped_cp(1).start()
    sp_cp(0).start()
    pltpu.sync_copy(table_hbm, tbl_v)

    for s in range(STEPS):
        b = s % NBUF
        ped_cp(s).wait()
        if s + 1 < STEPS:
            if s - 2 >= 0:
                out_cp(s - 2).wait()
            sp_cp(s + 1).start()
        if s + 2 < STEPS:
            ped_cp(s + 2).start()

        ped_b = pv[b]
        out_b = ov[b]

        @plsc.parallel_loop(0, BLOCKS, unroll=4)
        def blk_fn(blk):
            sg = lax.shift_right_logical(blk, 3)
            k = jnp.bitwise_and(blk, KPG - 1)
            ped_vec = ped_b[pl.ds(blk * L, L)]
            for j in range(PED_F):
                cj = jnp.full((L,), j, jnp.int32)
                col = plsc.load_gather(tbl_v, [cj, ped_vec])
                out_b[sg, j, pl.ds(k * L, L)] = col

        sp_cp(s).wait()
        out_cp(s).start()

    for s in range(max(0, STEPS - NBUF), STEPS):
        out_cp(s).wait()


def kernel(pedestal_table, spatial_embeddings, pedestals):
    mesh = plsc.VectorSubcoreMesh(core_axis_name="c", subcore_axis_name="s")
    tblT = jnp.pad(pedestal_table.T, ((0, 0), (0, G - 16)))
    sp3 = spatial_embeddings.reshape(NG, G, SPA_F).transpose(0, 2, 1)
    scratch = (
        [pltpu.VMEM((PED_F, G), jnp.float32)]
        + [pltpu.VMEM((S,), jnp.int32) for _ in range(NBUF)]
        + [pltpu.VMEM((SG, 8, G), jnp.float32) for _ in range(NBUF)]
        + [pltpu.SemaphoreType.DMA for _ in range(3 * NBUF)]
    )
    k = functools.partial(
        pl.kernel,
        mesh=mesh,
        out_type=jax.ShapeDtypeStruct((NG, 8, G), jnp.float32),
        scratch_types=scratch,
        compiler_params=pltpu.CompilerParams(
            needs_layout_passes=False,
            use_tc_tiling_on_sc=False,
            skip_device_barrier=True,
            disable_bounds_checks=True,
            disable_semaphore_checks=True,
        ),
    )(_body)
    out3 = k(tblT, sp3, pedestals)
    return out3.transpose(0, 2, 1).reshape(N, 8)[:, :OUT_F]
